# baseline (speedup 1.0000x reference)
import jax
import jax.numpy as jnp
from jax.experimental import pallas as pl
from jax.experimental.pallas import tpu as pltpu

_CH = 512
_NBUF = 8


def _mm_mask_kernel(emb_hbm, mask_ref, w_ref, out_ref, bufs, sems):
    i = pl.program_id(0)
    nch = pl.num_programs(0)

    @pl.when(i == 0)
    def _prologue():
        for b in range(_NBUF):
            pltpu.make_async_copy(
                emb_hbm.at[pl.ds(b * _CH, _CH), :], bufs.at[b], sems.at[b]
            ).start()

    buf = jax.lax.rem(i, _NBUF)
    pltpu.make_async_copy(
        emb_hbm.at[pl.ds(i * _CH, _CH), :], bufs.at[buf], sems.at[buf]
    ).wait()

    e = bufs[buf]
    w = w_ref[...]
    logits = jax.lax.dot_general(
        e, w,
        dimension_numbers=(((1,), (1,)), ((), ())),
        preferred_element_type=jnp.float32,
    )
    m = mask_ref[...]
    out_ref[...] = jnp.where(m > 0, logits, -jnp.inf)

    nxt = i + _NBUF

    @pl.when(nxt < nch)
    def _next():
        pltpu.make_async_copy(
            emb_hbm.at[pl.ds(nxt * _CH, _CH), :], bufs.at[buf], sems.at[buf]
        ).start()


def kernel(emb_sentences, att_sentences, W):
    B, L, D = emb_sentences.shape
    NL = W.shape[0]
    R = B * L
    emb = emb_sentences.reshape(R, D)
    mask = att_sentences.reshape(R, 1).astype(jnp.float32)

    out = pl.pallas_call(
        _mm_mask_kernel,
        grid=(R // _CH,),
        in_specs=[
            pl.BlockSpec(memory_space=pl.ANY),
            pl.BlockSpec((_CH, 1), lambda i: (i, 0)),
            pl.BlockSpec((NL, D), lambda i: (0, 0)),
        ],
        out_specs=pl.BlockSpec((_CH, NL), lambda i: (i, 0)),
        out_shape=jax.ShapeDtypeStruct((R, NL), jnp.float32),
        scratch_shapes=[
            pltpu.VMEM((_NBUF, _CH, D), jnp.float32),
            pltpu.SemaphoreType.DMA((_NBUF,)),
        ],
    )(emb, mask, W)
    return out.reshape(B, L, NL)


# speedup vs baseline: 1.4436x; 1.4436x over previous
import jax
import jax.numpy as jnp
from jax.experimental import pallas as pl
from jax.experimental.pallas import tpu as pltpu


def _mm_mask_kernel(emb_ref, att_ref, w_ref, out_ref):
    e = emb_ref[0]
    w = w_ref[...]
    logits = jax.lax.dot_general(
        e, w,
        dimension_numbers=(((1,), (1,)), ((), ())),
        preferred_element_type=jnp.float32,
    )
    att = att_ref[0]
    out_ref[0] = jnp.where(att, logits, -jnp.inf)


def kernel(emb_sentences, att_sentences, W):
    B, L, D = emb_sentences.shape
    NL = W.shape[0]
    att3 = att_sentences.reshape(B, L, 1)

    return pl.pallas_call(
        _mm_mask_kernel,
        grid=(B,),
        in_specs=[
            pl.BlockSpec((1, L, D), lambda i: (i, 0, 0)),
            pl.BlockSpec((1, L, 1), lambda i: (i, 0, 0)),
            pl.BlockSpec((NL, D), lambda i: (0, 0)),
        ],
        out_specs=pl.BlockSpec((1, L, NL), lambda i: (i, 0, 0)),
        out_shape=jax.ShapeDtypeStruct((B, L, NL), jnp.float32),
        compiler_params=pltpu.CompilerParams(
            dimension_semantics=("parallel",),
        ),
    )(emb_sentences, att3, W)
